# baseline (speedup 1.0000x reference)
import functools

import jax
import jax.numpy as jnp
from jax import lax
from jax.experimental import pallas as pl
from jax.experimental.pallas import tpu as pltpu
from jax.experimental.pallas import tpu_sc as plsc

MAXP = 128
SEQ = 2048
D = 64
NC = 2
NS = 16
L = 16
NW = NC * NS
ROWS_PER_W = SEQ // NW
HALF = SEQ // 2
WND = HALF + ROWS_PER_W - 1
GCHUNK = 128
NG = (WND + GCHUNK - 1) // GCHUNK


def _sc_body(table_hbm, out_hbm, idx_v, wnd_v, gsem, wsem):
    cid = lax.axis_index("c")
    sid = lax.axis_index("s")
    i0 = (sid * NC + cid) * ROWS_PER_W

    def do_half(h, carry):
        s = (SEQ - 1) - (i0 + ROWS_PER_W - 1) + h * HALF

        def fill(k, carry):
            m = s + k * L + lax.iota(jnp.int32, L)
            r = MAXP - jnp.clip(m - (SEQ - 1), -MAXP, MAXP)
            idx_v[k // (GCHUNK // L), pl.ds(k % (GCHUNK // L) * L, L)] = r
            return carry

        lax.fori_loop(0, NG * GCHUNK // L, fill, 0)

        def gather(c, carry):
            pltpu.make_async_copy(
                table_hbm.at[idx_v.at[c]],
                wnd_v.at[pl.ds(c * GCHUNK, GCHUNK)],
                gsem,
            ).start()
            return carry

        lax.fori_loop(0, NG, gather, 0)

        def gwait(c, carry):
            pltpu.make_async_copy(
                table_hbm.at[idx_v.at[0]],
                wnd_v.at[pl.ds(0, GCHUNK)],
                gsem,
            ).wait()
            return carry

        lax.fori_loop(0, NG, gwait, 0)

        def emit(k, carry):
            pltpu.make_async_copy(
                wnd_v.at[pl.ds(ROWS_PER_W - 1 - k, HALF)],
                out_hbm.at[i0 + k, pl.ds(h * HALF, HALF)],
                wsem,
            ).start()
            return carry

        lax.fori_loop(0, ROWS_PER_W, emit, 0)

        def drain(k, carry):
            pltpu.make_async_copy(
                wnd_v.at[pl.ds(0, HALF)],
                out_hbm.at[i0, pl.ds(0, HALF)],
                wsem,
            ).wait()
            return carry

        lax.fori_loop(0, ROWS_PER_W, drain, 0)
        return carry

    lax.fori_loop(0, 2, do_half, 0)


_round_up = (WND + GCHUNK - 1) // GCHUNK * GCHUNK

_sc_call = functools.partial(
    pl.kernel,
    mesh=plsc.VectorSubcoreMesh(core_axis_name="c", subcore_axis_name="s"),
    out_type=jax.ShapeDtypeStruct((SEQ, SEQ, D), jnp.float32),
    scratch_types=[
        pltpu.VMEM((NG, GCHUNK), jnp.int32),
        pltpu.VMEM((_round_up, D), jnp.float32),
        pltpu.SemaphoreType.DMA,
        pltpu.SemaphoreType.DMA,
    ],
    compiler_params=pltpu.CompilerParams(use_tc_tiling_on_sc=False),
)(_sc_body)


def kernel(i_indices, j_indices, embeddings_table):
    return _sc_call(embeddings_table)


# speedup vs baseline: 1.3570x; 1.3570x over previous
import functools

import jax
import jax.numpy as jnp
from jax import lax
from jax.experimental import pallas as pl
from jax.experimental.pallas import tpu as pltpu
from jax.experimental.pallas import tpu_sc as plsc

MAXP = 128
SEQ = 2048
D = 64
HPAD = 2 * SEQ
NC = 2
NS = 16
L = 16
NW = NC * NS
ROWS_PER_W = SEQ // NW
BUILD_PER_S = HPAD // NS
GCHUNK = 128


def _sc_body(table_hbm, out_hbm, idx_v, rows_v, h_sh, sem, out_sem):
    cid = lax.axis_index("c")
    sid = lax.axis_index("s")

    base = sid * BUILD_PER_S
    for g in range(BUILD_PER_S // GCHUNK):
        gbase = base + g * GCHUNK

        def fill(k, carry, gbase=gbase):
            m = gbase + k * L + lax.iota(jnp.int32, L)
            r = MAXP - jnp.clip(m - (SEQ - 1), -MAXP, MAXP)
            idx_v[pl.ds(k * L, L)] = r
            return carry

        lax.fori_loop(0, GCHUNK // L, fill, 0)
        pltpu.async_copy(table_hbm.at[idx_v], rows_v, sem).wait()
        pltpu.sync_copy(rows_v, h_sh.at[pl.ds(gbase, GCHUNK)])
    plsc.subcore_barrier()

    row0 = (sid * NC + cid) * ROWS_PER_W

    def emit(k, carry):
        i = row0 + k
        pltpu.make_async_copy(
            h_sh.at[pl.ds((SEQ - 1) - i, SEQ)], out_hbm.at[i], out_sem
        ).start()
        return carry

    lax.fori_loop(0, ROWS_PER_W, emit, 0)

    def drain(k, carry):
        i = row0 + k
        pltpu.make_async_copy(
            h_sh.at[pl.ds((SEQ - 1) - i, SEQ)], out_hbm.at[i], out_sem
        ).wait()
        return carry

    lax.fori_loop(0, ROWS_PER_W, drain, 0)


_sc_call = functools.partial(
    pl.kernel,
    mesh=plsc.VectorSubcoreMesh(core_axis_name="c", subcore_axis_name="s"),
    out_type=jax.ShapeDtypeStruct((SEQ, SEQ, D), jnp.float32),
    scratch_types=[
        pltpu.VMEM((GCHUNK,), jnp.int32),
        pltpu.VMEM((GCHUNK, D), jnp.float32),
        pltpu.VMEM_SHARED((HPAD, D), jnp.float32),
        pltpu.SemaphoreType.DMA,
        pltpu.SemaphoreType.DMA,
    ],
    compiler_params=pltpu.CompilerParams(use_tc_tiling_on_sc=False),
)(_sc_body)


def kernel(i_indices, j_indices, embeddings_table):
    return _sc_call(embeddings_table)


# speedup vs baseline: 4.1941x; 3.0908x over previous
import functools

import jax
import jax.numpy as jnp
from jax import lax
from jax.experimental import pallas as pl
from jax.experimental.pallas import tpu as pltpu
from jax.experimental.pallas import tpu_sc as plsc

MAXP = 128
SEQ = 2048
D = 64
HPAD = 2 * SEQ
NC = 2
NS = 16
L = 16
NW = NC * NS
K_PER_W = D // NW


def _sc_body(tabf_hbm, ht_hbm, tab_v, buf_v):
    wid = lax.axis_index("s") * NC + lax.axis_index("c")
    pltpu.sync_copy(tabf_hbm, tab_v)
    k0 = wid * K_PER_W
    for kk in range(K_PER_W):
        def fill(v, carry, kk=kk):
            m = v * L + lax.iota(jnp.int32, L)
            rr = MAXP - jnp.clip(m - (SEQ - 1), -MAXP, MAXP)
            buf_v[kk, pl.ds(v * L, L)] = plsc.load_gather(
                tab_v, [rr * D + (k0 + kk)]
            )
            return carry

        lax.fori_loop(0, HPAD // L, fill, 0)
    pltpu.sync_copy(buf_v, ht_hbm.at[pl.ds(k0, K_PER_W)])


_sc_build_ht = functools.partial(
    pl.kernel,
    mesh=plsc.VectorSubcoreMesh(core_axis_name="c", subcore_axis_name="s"),
    out_type=jax.ShapeDtypeStruct((D, HPAD), jnp.float32),
    scratch_types=[
        pltpu.VMEM(((2 * MAXP + 1) * D,), jnp.float32),
        pltpu.VMEM((K_PER_W, HPAD), jnp.float32),
    ],
    compiler_params=pltpu.CompilerParams(
        use_tc_tiling_on_sc=False, needs_layout_passes=False
    ),
)(_sc_body)


def _tc_body(ht_ref, out_ref, r_scr):
    rho = pl.program_id(0)
    t = pl.program_id(1)

    @pl.when(t == 0)
    def _():
        r_scr[...] = pltpu.roll(ht_ref[...], HPAD - rho, axis=1)

    start = pl.multiple_of(t * 128, 128)
    out_ref[0] = r_scr[:, pl.ds(start, SEQ)]


_tc_stream = pl.pallas_call(
    _tc_body,
    grid=(128, SEQ // 128),
    in_specs=[pl.BlockSpec((D, HPAD), lambda i, j: (0, 0))],
    out_specs=pl.BlockSpec(
        (1, D, SEQ), lambda i, j: (SEQ - 1 - i - 128 * j, 0, 0)
    ),
    out_shape=jax.ShapeDtypeStruct((SEQ, D, SEQ), jnp.float32),
    scratch_shapes=[pltpu.VMEM((D, HPAD), jnp.float32)],
    compiler_params=pltpu.CompilerParams(
        dimension_semantics=("arbitrary", "arbitrary")
    ),
)


def kernel(i_indices, j_indices, embeddings_table):
    ht = _sc_build_ht(embeddings_table.reshape(-1))
    out_t = _tc_stream(ht)
    return jnp.swapaxes(out_t, 1, 2)


# speedup vs baseline: 11.6431x; 2.7760x over previous
import functools

import jax
import jax.numpy as jnp
from jax import lax
from jax.experimental import pallas as pl
from jax.experimental.pallas import tpu as pltpu
from jax.experimental.pallas import tpu_sc as plsc

MAXP = 128
SEQ = 2048
D = 64
HPAD = 2 * SEQ
NC = 2
NS = 16
L = 16
NW = NC * NS
K_PER_W = D // NW


def _sc_body(tabf_hbm, ht_hbm, tab_v, buf_v):
    wid = lax.axis_index("s") * NC + lax.axis_index("c")
    pltpu.sync_copy(tabf_hbm, tab_v)
    k0 = wid * K_PER_W
    for kk in range(K_PER_W):
        def fill(v, carry, kk=kk):
            m = v * L + lax.iota(jnp.int32, L)
            rr = MAXP - jnp.clip(m - (SEQ - 1), -MAXP, MAXP)
            buf_v[kk, pl.ds(v * L, L)] = plsc.load_gather(
                tab_v, [rr * D + (k0 + kk)]
            )
            return carry

        lax.fori_loop(0, HPAD // L, fill, 0)
    pltpu.sync_copy(buf_v, ht_hbm.at[pl.ds(k0, K_PER_W)])


_sc_build_ht = functools.partial(
    pl.kernel,
    mesh=plsc.VectorSubcoreMesh(core_axis_name="c", subcore_axis_name="s"),
    out_type=jax.ShapeDtypeStruct((D, HPAD), jnp.float32),
    scratch_types=[
        pltpu.VMEM(((2 * MAXP + 1) * D,), jnp.float32),
        pltpu.VMEM((K_PER_W, HPAD), jnp.float32),
    ],
    compiler_params=pltpu.CompilerParams(
        use_tc_tiling_on_sc=False, needs_layout_passes=False
    ),
)(_sc_body)


_NT = SEQ // 128


def _tc_body(ht_ref, out_ref, buf, sem):
    rho = pl.program_id(0)
    t = pl.program_id(1)
    b = lax.rem(rho, 2)

    def drain(n):
        def body(_, carry):
            pltpu.make_async_copy(
                buf.at[0, :, pl.ds(0, SEQ)], out_ref.at[0], sem
            ).wait()
            return carry

        lax.fori_loop(0, n, body, 0)

    @pl.when(t == 0)
    def _():
        @pl.when(rho >= 2)
        def _():
            drain(_NT)

        buf[b] = pltpu.roll(ht_ref[...], HPAD - rho, axis=1)

    i = SEQ - 1 - rho - 128 * t
    start = pl.multiple_of(t * 128, 128)
    pltpu.make_async_copy(
        buf.at[b, :, pl.ds(start, SEQ)], out_ref.at[i], sem
    ).start()

    @pl.when((rho == 127) & (t == _NT - 1))
    def _():
        drain(2 * _NT)


_tc_stream = pl.pallas_call(
    _tc_body,
    grid=(128, _NT),
    in_specs=[pl.BlockSpec((D, HPAD), lambda i, j: (0, 0))],
    out_specs=pl.BlockSpec(memory_space=pl.ANY),
    out_shape=jax.ShapeDtypeStruct((SEQ, D, SEQ), jnp.float32),
    scratch_shapes=[
        pltpu.VMEM((2, D, HPAD), jnp.float32),
        pltpu.SemaphoreType.DMA,
    ],
    compiler_params=pltpu.CompilerParams(
        dimension_semantics=("arbitrary", "arbitrary")
    ),
)


def kernel(i_indices, j_indices, embeddings_table):
    ht = _sc_build_ht(embeddings_table.reshape(-1))
    out_t = _tc_stream(ht)
    return jnp.swapaxes(out_t, 1, 2)
